# baseline (speedup 1.0000x reference)
import functools
import jax
import jax.numpy as jnp
from jax import lax
from jax.experimental import pallas as pl
from jax.experimental.pallas import tpu as pltpu
from jax.experimental.pallas import tpu_sc as plsc

BS = 128
VOCAB = 100000
LANES = 16
NGRP = BS // LANES
STRIPE = 3200
CROWS = 200
NCHUNK = STRIPE // CROWS
QUART = CROWS // 4

NEG_BIG = -3.0e38
IDX_BIG = 2 ** 30

_info = plsc.get_sparse_core_info()
NC = _info.num_cores
NS = _info.num_subcores
NW = NC * NS
MAX_OFF8 = (VOCAB - STRIPE) // 8

_mesh = plsc.VectorSubcoreMesh(core_axis_name="c", subcore_axis_name="s")


@functools.partial(
    pl.kernel,
    mesh=_mesh,
    out_type=(
        jax.ShapeDtypeStruct((NW * BS,), jnp.float32),
        jax.ShapeDtypeStruct((NW * BS,), jnp.int32),
    ),
    scratch_types=[
        pltpu.VMEM((2, CROWS, BS), jnp.float32),
        pltpu.VMEM((2, CROWS, BS), jnp.int32),
        pltpu.VMEM((BS,), jnp.float32),
        pltpu.VMEM((BS,), jnp.float32),
        pltpu.VMEM((BS,), jnp.int32),
        pltpu.SemaphoreType.DMA,
        pltpu.SemaphoreType.DMA,
        pltpu.SemaphoreType.DMA,
        pltpu.SemaphoreType.DMA,
    ],
)
def _sc_scan(l_hbm, t_hbm, pen_hbm, val_hbm, idx_hbm,
             lbuf, tbuf, penv, pv, pi,
             lsem0, lsem1, tsem0, tsem1):
    scid = lax.axis_index("c")
    sidx = lax.axis_index("s")
    wid = sidx * NC + scid
    off = pl.multiple_of((wid * MAX_OFF8) // (NW - 1) * 8, 8)
    lsems = (lsem0, lsem1)
    tsems = (tsem0, tsem1)

    pltpu.sync_copy(pen_hbm, penv)

    def lsrc(c):
        return l_hbm.at[pl.ds(off + c * CROWS, CROWS), pl.ds(0, BS)]

    def tsrc(c):
        return t_hbm.at[pl.ds(off + c * CROWS, CROWS), pl.ds(0, BS)]

    def start(c, b):
        pltpu.async_copy(lsrc(c), lbuf.at[b], lsems[b])
        pltpu.async_copy(tsrc(c), tbuf.at[b], tsems[b])

    def wait(c, b):
        pltpu.make_async_copy(lsrc(c), lbuf.at[b], lsems[b]).wait()
        pltpu.make_async_copy(tsrc(c), tbuf.at[b], tsems[b]).wait()

    pens = [penv[pl.ds(g * LANES, LANES)] for g in range(NGRP)]
    rps = [1.0 / p for p in pens]

    start(0, 0)

    def chunk_body(gc, carry):
        accs = list(carry)
        for b in range(2):
            c = gc * 2 + b
            wait(c, b)

            @pl.when(c + 1 < NCHUNK)
            def _():
                start(c + 1, 1 - b)

            base = off + c * CROWS
            for g in range(NGRP):
                pen_s = pens[g]
                rp_s = rps[g]

                def vbody(j, a, b=b, g=g, pen_s=pen_s, rp_s=rp_s, base=base):
                    col = g * LANES
                    out = []
                    for q in range(4):
                        v, i = a[2 * q], a[2 * q + 1]
                        r = j + q * QUART
                        l = lbuf[b, r, pl.ds(col, LANES)]
                        t = tbuf[b, r, pl.ds(col, LANES)]
                        p = jnp.where(t > 0,
                                      jnp.minimum(l * rp_s, l * pen_s), l)
                        ix = jnp.full((LANES,), base + r, jnp.int32)
                        up = p > v
                        out.append(jnp.where(up, p, v))
                        out.append(jnp.where(up, ix, i))
                    return tuple(out)

                accs[g] = lax.fori_loop(0, QUART, vbody, accs[g])
        return tuple(accs)

    acc0 = []
    for _g in range(NGRP):
        one = []
        for _q in range(4):
            one.append(jnp.full((LANES,), NEG_BIG, jnp.float32))
            one.append(jnp.full((LANES,), 0, jnp.int32))
        acc0.append(tuple(one))
    accs = lax.fori_loop(0, NCHUNK // 2, chunk_body, tuple(acc0))

    for g in range(NGRP):
        a = accs[g]
        bv, bi = a[0], a[1]
        for q in range(1, 4):
            v2, i2 = a[2 * q], a[2 * q + 1]
            up = jnp.logical_or(v2 > bv, jnp.logical_and(v2 == bv, i2 < bi))
            bv = jnp.where(up, v2, bv)
            bi = jnp.where(up, i2, bi)
        pv[pl.ds(g * LANES, LANES)] = bv
        pi[pl.ds(g * LANES, LANES)] = bi

    pltpu.sync_copy(pv, val_hbm.at[pl.ds(wid * BS, BS)])
    pltpu.sync_copy(pi, idx_hbm.at[pl.ds(wid * BS, BS)])


def _merge_body(v_ref, i_ref, o_ref):
    v = v_ref[...]
    i = i_ref[...]
    m = jnp.max(v, axis=0, keepdims=True)
    cand = jnp.where(v == m, i, IDX_BIG)
    o_ref[...] = jnp.min(cand, axis=0, keepdims=True)


def kernel(logits, repetition_penalty, token_count):
    lt = logits.reshape(BS, VOCAB).T
    tt = token_count.T
    pen = repetition_penalty.reshape(BS)
    vals, idxs = _sc_scan(lt, tt, pen)
    out = pl.pallas_call(
        _merge_body,
        out_shape=jax.ShapeDtypeStruct((1, BS), jnp.int32),
    )(vals.reshape(NW, BS), idxs.reshape(NW, BS))
    return out.reshape(BS, 1)


# speedup vs baseline: 1.3148x; 1.3148x over previous
import functools
import jax
import jax.numpy as jnp
from jax import lax
from jax.experimental import pallas as pl
from jax.experimental.pallas import tpu as pltpu
from jax.experimental.pallas import tpu_sc as plsc

BS = 128
VOCAB = 100000
LANES = 16
NGRP = BS // LANES

VTC = 64000
VB = 8000
NTCBLK = VTC // VB

SCBASE = VTC
SCLEN = VOCAB - VTC
STRIPE = 1200
CROWS = 200
NCHUNK = STRIPE // CROWS
QUART = CROWS // 4

NEG_BIG = -3.0e38
IDX_BIG = 2 ** 30

_info = plsc.get_sparse_core_info()
NC = _info.num_cores
NS = _info.num_subcores
NW = NC * NS
MAX_OFF8 = (SCLEN - STRIPE) // 8

_mesh = plsc.VectorSubcoreMesh(core_axis_name="c", subcore_axis_name="s")


@functools.partial(
    pl.kernel,
    mesh=_mesh,
    out_type=(
        jax.ShapeDtypeStruct((NW * BS,), jnp.float32),
        jax.ShapeDtypeStruct((NW * BS,), jnp.int32),
    ),
    scratch_types=[
        pltpu.VMEM((2, CROWS, BS), jnp.float32),
        pltpu.VMEM((2, CROWS, BS), jnp.int32),
        pltpu.VMEM((BS,), jnp.float32),
        pltpu.VMEM((BS,), jnp.float32),
        pltpu.VMEM((BS,), jnp.int32),
        pltpu.SemaphoreType.DMA,
        pltpu.SemaphoreType.DMA,
        pltpu.SemaphoreType.DMA,
        pltpu.SemaphoreType.DMA,
    ],
)
def _sc_scan(l_hbm, t_hbm, pen_hbm, val_hbm, idx_hbm,
             lbuf, tbuf, penv, pv, pi,
             lsem0, lsem1, tsem0, tsem1):
    scid = lax.axis_index("c")
    sidx = lax.axis_index("s")
    wid = sidx * NC + scid
    off = pl.multiple_of(SCBASE + (wid * MAX_OFF8) // (NW - 1) * 8, 8)
    lsems = (lsem0, lsem1)
    tsems = (tsem0, tsem1)

    pltpu.sync_copy(pen_hbm, penv)

    def lsrc(c):
        return l_hbm.at[pl.ds(off + c * CROWS, CROWS), pl.ds(0, BS)]

    def tsrc(c):
        return t_hbm.at[pl.ds(off + c * CROWS, CROWS), pl.ds(0, BS)]

    def start(c, b):
        pltpu.async_copy(lsrc(c), lbuf.at[b], lsems[b])
        pltpu.async_copy(tsrc(c), tbuf.at[b], tsems[b])

    def wait(c, b):
        pltpu.make_async_copy(lsrc(c), lbuf.at[b], lsems[b]).wait()
        pltpu.make_async_copy(tsrc(c), tbuf.at[b], tsems[b]).wait()

    pens = [penv[pl.ds(g * LANES, LANES)] for g in range(NGRP)]
    rps = [1.0 / p for p in pens]

    start(0, 0)

    def chunk_body(gc, carry):
        accs = list(carry)
        for b in range(2):
            c = gc * 2 + b
            wait(c, b)

            @pl.when(c + 1 < NCHUNK)
            def _():
                start(c + 1, 1 - b)

            base = off + c * CROWS
            for g in range(NGRP):
                pen_s = pens[g]
                rp_s = rps[g]

                def vbody(j, a, b=b, g=g, pen_s=pen_s, rp_s=rp_s, base=base):
                    col = g * LANES
                    out = []
                    for q in range(4):
                        v, i = a[2 * q], a[2 * q + 1]
                        r = j + q * QUART
                        l = lbuf[b, r, pl.ds(col, LANES)]
                        t = tbuf[b, r, pl.ds(col, LANES)]
                        p = jnp.where(t > 0,
                                      jnp.minimum(l * rp_s, l * pen_s), l)
                        ix = jnp.full((LANES,), base + r, jnp.int32)
                        up = p > v
                        out.append(jnp.where(up, p, v))
                        out.append(jnp.where(up, ix, i))
                    return tuple(out)

                accs[g] = lax.fori_loop(0, QUART, vbody, accs[g])
        return tuple(accs)

    acc0 = []
    for _g in range(NGRP):
        one = []
        for _q in range(4):
            one.append(jnp.full((LANES,), NEG_BIG, jnp.float32))
            one.append(jnp.full((LANES,), 0, jnp.int32))
        acc0.append(tuple(one))
    accs = lax.fori_loop(0, NCHUNK // 2, chunk_body, tuple(acc0))

    for g in range(NGRP):
        a = accs[g]
        bv, bi = a[0], a[1]
        for q in range(1, 4):
            v2, i2 = a[2 * q], a[2 * q + 1]
            up = jnp.logical_or(v2 > bv, jnp.logical_and(v2 == bv, i2 < bi))
            bv = jnp.where(up, v2, bv)
            bi = jnp.where(up, i2, bi)
        pv[pl.ds(g * LANES, LANES)] = bv
        pi[pl.ds(g * LANES, LANES)] = bi

    pltpu.sync_copy(pv, val_hbm.at[pl.ds(wid * BS, BS)])
    pltpu.sync_copy(pi, idx_hbm.at[pl.ds(wid * BS, BS)])


def _tc_body(l_ref, t_ref, p_ref, ov_ref, oi_ref, accv, acci):
    k = pl.program_id(0)
    pen = p_ref[...]
    rp = 1.0 / pen
    l = l_ref[...]
    t = t_ref[...]
    p = jnp.where(t > 0, jnp.minimum(l * rp, l * pen), l)
    p3 = p.reshape(VB // 8, 8, BS)
    idx3 = (k * VB
            + jax.lax.broadcasted_iota(jnp.int32, (VB // 8, 8, BS), 0) * 8
            + jax.lax.broadcasted_iota(jnp.int32, (VB // 8, 8, BS), 1))
    bm = jnp.max(p3, axis=0)
    cand = jnp.where(p3 == bm[None], idx3, IDX_BIG)
    bi = jnp.min(cand, axis=0)

    @pl.when(k == 0)
    def _():
        accv[...] = bm
        acci[...] = bi

    @pl.when(k > 0)
    def _():
        av = accv[...]
        up = bm > av
        accv[...] = jnp.where(up, bm, av)
        acci[...] = jnp.where(up, bi, acci[...])

    @pl.when(k == NTCBLK - 1)
    def _():
        av = accv[...]
        ai = acci[...]
        m = jnp.max(av, axis=0, keepdims=True)
        c2 = jnp.where(av == m, ai, IDX_BIG)
        ov_ref[...] = m
        oi_ref[...] = jnp.min(c2, axis=0, keepdims=True)


def _merge_body(v_ref, i_ref, tv_ref, ti_ref, o_ref):
    v = v_ref[...]
    i = i_ref[...]
    tv = tv_ref[...]
    ti = ti_ref[...]
    m1 = jnp.max(v, axis=0, keepdims=True)
    m = jnp.maximum(m1, tv)
    c_sc = jnp.min(jnp.where(v == m, i, IDX_BIG), axis=0, keepdims=True)
    c_tc = jnp.where(tv == m, ti, IDX_BIG)
    o_ref[...] = jnp.minimum(c_sc, c_tc)


def kernel(logits, repetition_penalty, token_count):
    lt = logits.reshape(BS, VOCAB).T
    tt = token_count.T
    pen = repetition_penalty.reshape(BS)
    pen2 = repetition_penalty.reshape(1, BS)
    vals, idxs = _sc_scan(lt, tt, pen)
    tcv, tci = pl.pallas_call(
        _tc_body,
        grid=(NTCBLK,),
        in_specs=[
            pl.BlockSpec((VB, BS), lambda k: (k, 0)),
            pl.BlockSpec((VB, BS), lambda k: (k, 0)),
            pl.BlockSpec((1, BS), lambda k: (0, 0)),
        ],
        out_specs=[
            pl.BlockSpec((1, BS), lambda k: (0, 0)),
            pl.BlockSpec((1, BS), lambda k: (0, 0)),
        ],
        out_shape=[
            jax.ShapeDtypeStruct((1, BS), jnp.float32),
            jax.ShapeDtypeStruct((1, BS), jnp.int32),
        ],
        scratch_shapes=[
            pltpu.VMEM((8, BS), jnp.float32),
            pltpu.VMEM((8, BS), jnp.int32),
        ],
    )(lt, tt, pen2)
    out = pl.pallas_call(
        _merge_body,
        out_shape=jax.ShapeDtypeStruct((1, BS), jnp.int32),
    )(vals.reshape(NW, BS), idxs.reshape(NW, BS), tcv, tci)
    return out.reshape(BS, 1)
